# baseline (speedup 1.0000x reference)
import jax
import jax.numpy as jnp
from jax.experimental import pallas as pl


def _splayer_copy_kernel(wav_ref, len_ref, wav_out_ref, len_out_ref):
    wav_out_ref[...] = wav_ref[...]
    len_out_ref[...] = len_ref[...]


def kernel(wav_batch, lengths):
    lengths_2d = jnp.asarray(lengths).astype(jnp.int32).reshape(1, lengths.shape[0])
    wav_out, len_out = pl.pallas_call(
        _splayer_copy_kernel,
        out_shape=(
            jax.ShapeDtypeStruct(wav_batch.shape, wav_batch.dtype),
            jax.ShapeDtypeStruct(lengths_2d.shape, jnp.int32),
        ),
    )(wav_batch, lengths_2d)
    return wav_out, len_out.reshape(lengths.shape)
